# baseline (speedup 1.0000x reference)
import functools

import jax
import jax.numpy as jnp
from jax import lax
from jax.experimental import pallas as pl
from jax.experimental.pallas import tpu as pltpu
from jax.experimental.pallas import tpu_sc as plsc

N = 50000
D = 128
P = 100000

NP = 50176
BA = 1792
GA = NP // BA
PPAD = 102400
PW = PPAD // 32
CH = 128
KCH = PW // CH
RT = NP // 16
DUMP = N + 64
BC = 3584
GC = NP // BC


def _sc_mesh():
    return plsc.VectorSubcoreMesh(core_axis_name="c", subcore_axis_name="s",
                                  num_cores=2, num_subcores=16)


def _lrelu(x):
    return jnp.where(x >= 0, x, 0.001 * x)


def _kh_body(idx_hbm, ones_hbm, zero_hbm, part_out, acc, idx_v, s_v, sem):
    cid = lax.axis_index("c")
    sid = lax.axis_index("s")
    wid = sid * 2 + cid
    r0 = sid * RT
    pltpu.sync_copy(zero_hbm.at[pl.ds(r0, RT)], acc.at[pl.ds(r0, RT)])
    pltpu.sync_copy(ones_hbm, s_v)
    plsc.subcore_barrier()
    for k in range(KCH):
        off = wid * PW + k * CH
        pltpu.sync_copy(idx_hbm.at[pl.ds(off, CH)], idx_v)
        pltpu.async_copy(s_v, acc.at[idx_v], sem, add=True).wait()
    plsc.subcore_barrier()
    pltpu.sync_copy(acc.at[pl.ds(r0, RT)], part_out.at[cid, pl.ds(r0, RT)])


def _hist(idx0, ones16, zero16):
    k = functools.partial(
        pl.kernel,
        out_type=jax.ShapeDtypeStruct((2, NP, 16), jnp.float32),
        mesh=_sc_mesh(),
        compiler_params=pltpu.CompilerParams(use_tc_tiling_on_sc=False),
        scratch_types=[
            pltpu.VMEM_SHARED((NP, 16), jnp.float32),
            pltpu.VMEM((CH,), jnp.int32),
            pltpu.VMEM((CH, 16), jnp.float32),
            pltpu.SemaphoreType.DMA,
        ],
    )(_kh_body)
    return k(idx0, ones16, zero16)


def _ka_body(enc, encnx, cp, cnx, wcat, w5, b1t,
             w2t, b2, w3t, b3, w4t, b4, gT):
    m = jnp.dot(enc[...], wcat[...], preferred_element_type=jnp.float32)
    mn = jnp.dot(encnx[0], wcat[...], preferred_element_type=jnp.float32)
    acc = m[:, 0:D]
    acc += jnp.concatenate([m[1:, D:2 * D], mn[0:1, D:2 * D]], axis=0)
    acc += jnp.concatenate([m[2:, 2 * D:3 * D], mn[0:2, 2 * D:3 * D]], axis=0)
    acc += jnp.concatenate([m[3:, 3 * D:4 * D], mn[0:3, 3 * D:4 * D]], axis=0)

    ct = jnp.transpose(cp[...].reshape(BA, 3))
    cn3 = jnp.transpose(cnx[0])
    c0 = ct
    c1 = jnp.concatenate([ct[:, 1:], cn3[:, :1]], axis=1)
    c2 = jnp.concatenate([ct[:, 2:], cn3[:, :2]], axis=1)
    c3 = jnp.concatenate([ct[:, 3:], cn3[:, :3]], axis=1)
    x0, y0, z0 = c0[0:1], c0[1:2], c0[2:3]
    x1, y1, z1 = c1[0:1], c1[1:2], c1[2:3]
    x2, y2, z2 = c2[0:1], c2[1:2], c2[2:3]
    x3, y3, z3 = c3[0:1], c3[1:2], c3[2:3]
    u1x, u1y, u1z = x1 - x0, y1 - y0, z1 - z0
    u2x, u2y, u2z = x2 - x1, y2 - y1, z2 - z1
    u3x, u3y, u3z = x3 - x2, y3 - y2, z3 - z2
    ax = u1y * u2z - u1z * u2y
    ay = u1z * u2x - u1x * u2z
    az = u1x * u2y - u1y * u2x
    bx = u2y * u3z - u2z * u3y
    by = u2z * u3x - u2x * u3z
    bz = u2x * u3y - u2y * u3x
    u2n = jnp.sqrt(u2x * u2x + u2y * u2y + u2z * u2z)
    ydot = u2n * (u1x * bx + u1y * by + u1z * bz)
    xdot = ax * bx + ay * by + az * bz
    rinv = lax.rsqrt(jnp.maximum(xdot * xdot + ydot * ydot, 1e-24))
    sin_t = ydot * rinv
    cos_t = xdot * rinv
    drx, dry, drz = x0 - x3, y0 - y3, z0 - z3
    dl2 = jnp.maximum(drx * drx + dry * dry + drz * drz, 1e-12)
    dlr = lax.rsqrt(dl2)
    dl = dl2 * dlr

    feat = jnp.concatenate(
        [sin_t, cos_t, dl, jnp.zeros((13, BA), jnp.float32)],
        axis=0)
    ft = jnp.transpose(feat)

    h = acc + jnp.dot(ft, w5[...], preferred_element_type=jnp.float32) \
        + b1t[...]
    h = _lrelu(h)
    h = _lrelu(jnp.dot(h, w2t[...], preferred_element_type=jnp.float32)
               + b2[...])
    h = _lrelu(jnp.dot(h, w3t[...], preferred_element_type=jnp.float32)
               + b3[...])
    dlt = jnp.dot(h, w4t[...], preferred_element_type=jnp.float32) + b4[...]

    dltT = jnp.transpose(dlt)
    d0 = -0.5 * dltT[0:1, :]
    d1 = 0.5 * dltT[1:2, :]
    dhx, dhy, dhz = drx * dlr, dry * dlr, drz * dlr
    gT[...] = jnp.concatenate(
        [d0 * dhx, d0 * dhy, d0 * dhz,
         d1 * dhx, d1 * dhy, d1 * dhz,
         jnp.zeros((2, BA), jnp.float32)], axis=0)


def _make_tables(encoded, encnx, coords, cnx, wcat, w5, b1t,
                 w2t, b2, w3t, b3, w4t, b4):
    full = lambda i: (0, 0)
    return pl.pallas_call(
        _ka_body,
        grid=(GA,),
        in_specs=[
            pl.BlockSpec((BA, D), lambda i: (i, 0)),
            pl.BlockSpec((1, 8, D), lambda i: (i, 0, 0)),
            pl.BlockSpec((BA, 1, 3), lambda i: (i, 0, 0)),
            pl.BlockSpec((1, 8, 3), lambda i: (i, 0, 0)),
            pl.BlockSpec((D, 4 * D), full),
            pl.BlockSpec((16, D), full),
            pl.BlockSpec((1, D), full),
            pl.BlockSpec((D, D), full),
            pl.BlockSpec((1, D), full),
            pl.BlockSpec((D, D), full),
            pl.BlockSpec((1, D), full),
            pl.BlockSpec((D, 16), full),
            pl.BlockSpec((1, 16), full),
        ],
        out_specs=pl.BlockSpec((8, BA), lambda i: (0, i)),
        out_shape=jax.ShapeDtypeStruct((8, NP), jnp.float32),
    )(encoded, encnx, coords, cnx, wcat, w5, b1t, w2t, b2, w3t, b3, w4t, b4)


def _ke_body(pa, pb, pap, pbp, gcur, gprv, ans, out):
    pid = pl.program_id(0)
    cnt = pa[0] + pb[0]
    c0 = cnt[:, 0:1]
    cprev_blk = pap[0][BC - 3:] + pbp[0][BC - 3:]
    cprev = jnp.where(pid > 0, cprev_blk, 0.0)
    cs = jnp.concatenate([cprev, cnt[:BC - 3]], axis=0)
    c3 = cs[:, 0:1]
    gc = gcur[...]
    gs = jnp.concatenate([gprv[:, BC - 3:], gc[:, :BC - 3]], axis=1)
    gcT = jnp.transpose(gc)
    gsT = jnp.transpose(gs)
    val3 = jnp.where(c0 > 0, c0 * gcT[:, 0:3], 0.0) \
        + jnp.where(c3 > 0, c3 * gsT[:, 3:6], 0.0)
    a3 = ans[...].reshape(BC, 3)
    out[...] = (a3 + val3).reshape(BC, 1, 3)


def _combine(part, gT, answer):
    prv = lambda i: jnp.maximum(i - 1, 0)
    return pl.pallas_call(
        _ke_body,
        grid=(GC,),
        in_specs=[
            pl.BlockSpec((1, BC, 16), lambda i: (0, i, 0)),
            pl.BlockSpec((1, BC, 16), lambda i: (1, i, 0)),
            pl.BlockSpec((1, BC, 16), lambda i: (0, prv(i), 0)),
            pl.BlockSpec((1, BC, 16), lambda i: (1, prv(i), 0)),
            pl.BlockSpec((8, BC), lambda i: (0, i)),
            pl.BlockSpec((8, BC), lambda i: (0, prv(i))),
            pl.BlockSpec((BC, 1, 3), lambda i: (i, 0, 0)),
        ],
        out_specs=pl.BlockSpec((BC, 1, 3), lambda i: (i, 0, 0)),
        out_shape=jax.ShapeDtypeStruct((N, 1, 3), jnp.float32),
    )(part, part, part, part, gT, gT, answer)


def kernel(coords, propers, encoded, t, answer, W1, b1, W2, b2, W3, b3,
           W4, b4):
    propers = propers.astype(jnp.int32)
    base = propers[:, 0]
    idx0 = jnp.pad(base, (0, PPAD - P), constant_values=DUMP)

    encnx = jnp.stack(
        [encoded[(i + 1) * BA:(i + 1) * BA + 8] for i in range(GA - 1)]
        + [encoded[N - 8:N]])
    coords3 = coords[:, 0, :]
    cnx = jnp.stack(
        [coords3[(i + 1) * BA:(i + 1) * BA + 8] for i in range(GA - 1)]
        + [coords3[N - 8:N]])

    wcat = jnp.concatenate([W1[:, 0:D].T, W1[:, D:2 * D].T,
                            W1[:, 2 * D:3 * D].T, W1[:, 3 * D:4 * D].T],
                           axis=1)
    b1t = (b1 + t[0] * W1[:, 4 * D])[None, :]
    w5 = jnp.concatenate(
        [W1[:, 4 * D + 1][None, :], W1[:, 4 * D + 2][None, :],
         W1[:, 4 * D + 3][None, :], jnp.zeros((13, D), jnp.float32)],
        axis=0)
    w2t = W2.T
    w3t = W3.T
    w4t = jnp.pad(W4.T, ((0, 0), (0, 14)))
    b4p = jnp.pad(b4, (0, 14))[None, :]
    b2r = b2[None, :]
    b3r = b3[None, :]

    ones16 = jnp.ones((CH, 16), jnp.float32)
    zero16 = jnp.zeros((NP, 16), jnp.float32)

    part = _hist(idx0, ones16, zero16)
    gT = _make_tables(encoded, encnx, coords, cnx, wcat, w5,
                      b1t, w2t, b2r, w3t, b3r, w4t, b4p)
    return _combine(part, gT, answer)


# speedup vs baseline: 2.4340x; 2.4340x over previous
import functools

import jax
import jax.numpy as jnp
from jax import lax
from jax.experimental import pallas as pl
from jax.experimental.pallas import tpu as pltpu
from jax.experimental.pallas import tpu_sc as plsc

N = 50000
D = 128
P = 100000

NP = 50176
BA = 1792
GA = NP // BA
PPAD = 102400
PW = PPAD // 32
CH = 128
KCH = PW // CH
RT = NP // 16
DUMP = N + 64
BC = 3584
GC = NP // BC


def _sc_mesh():
    return plsc.VectorSubcoreMesh(core_axis_name="c", subcore_axis_name="s",
                                  num_cores=2, num_subcores=16)


def _lrelu(x):
    return jnp.where(x >= 0, x, 0.001 * x)


def _kh_body(idx_hbm, ones_hbm, zero_hbm, part_out, acc, idx_v, s_v, sem):
    cid = lax.axis_index("c")
    sid = lax.axis_index("s")
    wid = sid * 2 + cid
    r0 = sid * RT
    pltpu.sync_copy(zero_hbm.at[pl.ds(r0, RT)], acc.at[pl.ds(r0, RT)])
    pltpu.sync_copy(ones_hbm, s_v)
    plsc.subcore_barrier()
    for k in range(KCH):
        off = wid * PW + k * CH
        pltpu.sync_copy(idx_hbm.at[pl.ds(off, CH)], idx_v)
        pltpu.async_copy(s_v, acc.at[idx_v], sem, add=True).wait()
    plsc.subcore_barrier()
    pltpu.sync_copy(acc.at[pl.ds(r0, RT)], part_out.at[cid, pl.ds(r0, RT)])


def _hist(idx0, ones16, zero16):
    k = functools.partial(
        pl.kernel,
        out_type=jax.ShapeDtypeStruct((2, NP, 16), jnp.float32),
        mesh=_sc_mesh(),
        compiler_params=pltpu.CompilerParams(use_tc_tiling_on_sc=False),
        scratch_types=[
            pltpu.VMEM_SHARED((NP, 16), jnp.float32),
            pltpu.VMEM((CH,), jnp.int32),
            pltpu.VMEM((CH, 16), jnp.float32),
            pltpu.SemaphoreType.DMA,
        ],
    )(_kh_body)
    return k(idx0, ones16, zero16)


def _ka_body(enc, encnx, cp, cnx, wcat, w5, b1t,
             w2t, b2, w3t, b3, w4t, b4, gT):
    m = jnp.dot(enc[...], wcat[...], preferred_element_type=jnp.float32)
    mn = jnp.dot(encnx[0], wcat[...], preferred_element_type=jnp.float32)
    acc = m[:, 0:D]
    acc += jnp.concatenate([m[1:, D:2 * D], mn[0:1, D:2 * D]], axis=0)
    acc += jnp.concatenate([m[2:, 2 * D:3 * D], mn[0:2, 2 * D:3 * D]], axis=0)
    acc += jnp.concatenate([m[3:, 3 * D:4 * D], mn[0:3, 3 * D:4 * D]], axis=0)

    ct = cp[...]
    cn3 = cnx[0]
    c0 = ct
    c1 = jnp.concatenate([ct[:, 1:], cn3[:, :1]], axis=1)
    c2 = jnp.concatenate([ct[:, 2:], cn3[:, :2]], axis=1)
    c3 = jnp.concatenate([ct[:, 3:], cn3[:, :3]], axis=1)
    x0, y0, z0 = c0[0:1], c0[1:2], c0[2:3]
    x1, y1, z1 = c1[0:1], c1[1:2], c1[2:3]
    x2, y2, z2 = c2[0:1], c2[1:2], c2[2:3]
    x3, y3, z3 = c3[0:1], c3[1:2], c3[2:3]
    u1x, u1y, u1z = x1 - x0, y1 - y0, z1 - z0
    u2x, u2y, u2z = x2 - x1, y2 - y1, z2 - z1
    u3x, u3y, u3z = x3 - x2, y3 - y2, z3 - z2
    ax = u1y * u2z - u1z * u2y
    ay = u1z * u2x - u1x * u2z
    az = u1x * u2y - u1y * u2x
    bx = u2y * u3z - u2z * u3y
    by = u2z * u3x - u2x * u3z
    bz = u2x * u3y - u2y * u3x
    u2n = jnp.sqrt(u2x * u2x + u2y * u2y + u2z * u2z)
    ydot = u2n * (u1x * bx + u1y * by + u1z * bz)
    xdot = ax * bx + ay * by + az * bz
    rinv = lax.rsqrt(jnp.maximum(xdot * xdot + ydot * ydot, 1e-24))
    sin_t = ydot * rinv
    cos_t = xdot * rinv
    drx, dry, drz = x0 - x3, y0 - y3, z0 - z3
    dl2 = jnp.maximum(drx * drx + dry * dry + drz * drz, 1e-12)
    dlr = lax.rsqrt(dl2)
    dl = dl2 * dlr

    feat = jnp.concatenate(
        [sin_t, cos_t, dl, jnp.zeros((13, BA), jnp.float32)],
        axis=0)
    ft = jnp.transpose(feat)

    h = acc + jnp.dot(ft, w5[...], preferred_element_type=jnp.float32) \
        + b1t[...]
    h = _lrelu(h)
    h = _lrelu(jnp.dot(h, w2t[...], preferred_element_type=jnp.float32)
               + b2[...])
    h = _lrelu(jnp.dot(h, w3t[...], preferred_element_type=jnp.float32)
               + b3[...])
    dlt = jnp.dot(h, w4t[...], preferred_element_type=jnp.float32) + b4[...]

    dltT = jnp.transpose(dlt)
    d0 = -0.5 * dltT[0:1, :]
    d1 = 0.5 * dltT[1:2, :]
    dhx, dhy, dhz = drx * dlr, dry * dlr, drz * dlr
    gT[...] = jnp.concatenate(
        [d0 * dhx, d0 * dhy, d0 * dhz,
         d1 * dhx, d1 * dhy, d1 * dhz,
         jnp.zeros((2, BA), jnp.float32)], axis=0)


def _make_tables(encoded, encnx, cT, cnxT, wcat, w5, b1t,
                 w2t, b2, w3t, b3, w4t, b4):
    full = lambda i: (0, 0)
    return pl.pallas_call(
        _ka_body,
        grid=(GA,),
        in_specs=[
            pl.BlockSpec((BA, D), lambda i: (i, 0)),
            pl.BlockSpec((1, 8, D), lambda i: (i, 0, 0)),
            pl.BlockSpec((3, BA), lambda i: (0, i)),
            pl.BlockSpec((1, 3, 8), lambda i: (i, 0, 0)),
            pl.BlockSpec((D, 4 * D), full),
            pl.BlockSpec((16, D), full),
            pl.BlockSpec((1, D), full),
            pl.BlockSpec((D, D), full),
            pl.BlockSpec((1, D), full),
            pl.BlockSpec((D, D), full),
            pl.BlockSpec((1, D), full),
            pl.BlockSpec((D, 16), full),
            pl.BlockSpec((1, 16), full),
        ],
        out_specs=pl.BlockSpec((8, BA), lambda i: (0, i)),
        out_shape=jax.ShapeDtypeStruct((8, NP), jnp.float32),
    )(encoded, encnx, cT, cnxT, wcat, w5, b1t, w2t, b2, w3t, b3, w4t, b4)


def _ke_body(pa, pb, pap, pbp, gcur, gprv, ans, out):
    pid = pl.program_id(0)
    cnt = pa[0] + pb[0]
    c0 = cnt[0:1, :]
    cprev_blk = pap[0][0:1, BC - 3:] + pbp[0][0:1, BC - 3:]
    cprev = jnp.where(pid > 0, cprev_blk, 0.0)
    c3 = jnp.concatenate([cprev, c0[:, :BC - 3]], axis=1)
    gc = gcur[...]
    gs = jnp.concatenate([gprv[:, BC - 3:], gc[:, :BC - 3]], axis=1)
    val3 = jnp.where(c0 > 0, c0 * gc[0:3], 0.0) \
        + jnp.where(c3 > 0, c3 * gs[3:6], 0.0)
    out[...] = ans[...] + val3


def _combine(partT, gT, ansT):
    prv = lambda i: jnp.maximum(i - 1, 0)
    return pl.pallas_call(
        _ke_body,
        grid=(GC,),
        in_specs=[
            pl.BlockSpec((1, 16, BC), lambda i: (0, 0, i)),
            pl.BlockSpec((1, 16, BC), lambda i: (1, 0, i)),
            pl.BlockSpec((1, 16, BC), lambda i: (0, 0, prv(i))),
            pl.BlockSpec((1, 16, BC), lambda i: (1, 0, prv(i))),
            pl.BlockSpec((8, BC), lambda i: (0, i)),
            pl.BlockSpec((8, BC), lambda i: (0, prv(i))),
            pl.BlockSpec((3, BC), lambda i: (0, i)),
        ],
        out_specs=pl.BlockSpec((3, BC), lambda i: (0, i)),
        out_shape=jax.ShapeDtypeStruct((3, N), jnp.float32),
    )(partT, partT, partT, partT, gT, gT, ansT)


def kernel(coords, propers, encoded, t, answer, W1, b1, W2, b2, W3, b3,
           W4, b4):
    propers = propers.astype(jnp.int32)
    base = propers[:, 0]
    idx0 = jnp.pad(base, (0, PPAD - P), constant_values=DUMP)

    encnx = jnp.stack(
        [encoded[(i + 1) * BA:(i + 1) * BA + 8] for i in range(GA - 1)]
        + [encoded[N - 8:N]])
    cT = jnp.transpose(coords[:, 0, :])
    cnxT = jnp.stack(
        [cT[:, (i + 1) * BA:(i + 1) * BA + 8] for i in range(GA - 1)]
        + [cT[:, N - 8:N]])
    aT = jnp.transpose(answer[:, 0, :])

    wcat = jnp.concatenate([W1[:, 0:D].T, W1[:, D:2 * D].T,
                            W1[:, 2 * D:3 * D].T, W1[:, 3 * D:4 * D].T],
                           axis=1)
    b1t = (b1 + t[0] * W1[:, 4 * D])[None, :]
    w5 = jnp.concatenate(
        [W1[:, 4 * D + 1][None, :], W1[:, 4 * D + 2][None, :],
         W1[:, 4 * D + 3][None, :], jnp.zeros((13, D), jnp.float32)],
        axis=0)
    w2t = W2.T
    w3t = W3.T
    w4t = jnp.pad(W4.T, ((0, 0), (0, 14)))
    b4p = jnp.pad(b4, (0, 14))[None, :]
    b2r = b2[None, :]
    b3r = b3[None, :]

    ones16 = jnp.ones((CH, 16), jnp.float32)
    zero16 = jnp.zeros((NP, 16), jnp.float32)

    part = _hist(idx0, ones16, zero16)
    gT = _make_tables(encoded, encnx, cT, cnxT, wcat, w5,
                      b1t, w2t, b2r, w3t, b3r, w4t, b4p)
    partT = jnp.transpose(part, (0, 2, 1))
    outT = _combine(partT, gT, aT)
    return jnp.transpose(outT).reshape(N, 1, 3)
